# baseline (speedup 1.0000x reference)
import jax
import jax.numpy as jnp
from jax.experimental import pallas as pl
from jax.experimental.pallas import tpu as pltpu

E = 8
IN_LEN = 768
OUT_LEN = 768
EPAD = 128


def _moe_body(x_ref, wg_ref, wr_ref, wgam_ref, ins_ref, be_ref, we_ref, wb_ref,
              out_ref, a_scr, w_scr, gam_scr):
    e = pl.program_id(0)
    xf = x_ref[...]
    dn = (((1,), (1,)), ((), ()))

    @pl.when(e == 0)
    def _route():
        ins = ins_ref[...]
        mean_ins = jnp.mean(ins, axis=0, keepdims=True)
        rg = jax.lax.dot_general(mean_ins, wr_ref[...], dn,
                                 preferred_element_type=jnp.float32)
        gam_scr[...] = jax.lax.dot_general(mean_ins, wgam_ref[...], dn,
                                           preferred_element_type=jnp.float32)
        logits = jax.lax.dot_general(xf, wg_ref[...], dn,
                                     preferred_element_type=jnp.float32)
        logits = logits + rg
        col = jax.lax.broadcasted_iota(jnp.int32, logits.shape, 1)
        logits = jnp.where(col < E, logits, -jnp.inf)
        m = jnp.max(logits, axis=1, keepdims=True)
        s = jnp.sum(jnp.exp(logits - m), axis=1, keepdims=True)
        w_scr[...] = 1.0 / s
        a_scr[...] = jnp.argmax(logits, axis=1, keepdims=True).astype(jnp.int32)

    ins = ins_ref[...]
    mean_ins = jnp.mean(ins, axis=0, keepdims=True)
    mi8 = jnp.broadcast_to(mean_ins, (8, IN_LEN))
    beta8 = jax.lax.dot_general(mi8, wb_ref[0], dn,
                                preferred_element_type=jnp.float32)
    beta_row = beta8[0:1]
    lane = jax.lax.broadcasted_iota(jnp.int32, (1, EPAD), 1)
    gamma = jnp.sum(jnp.where(lane == e, gam_scr[...], 0.0))
    be_row = be_ref[pl.ds(e, 1), :]
    ce_row = gamma * be_row + beta_row

    comb_e = jnp.where(a_scr[...] == e, w_scr[...], 0.0)
    y = jax.lax.dot_general(xf.astype(jnp.bfloat16),
                            we_ref[0].astype(jnp.bfloat16), dn,
                            preferred_element_type=jnp.float32)
    contrib = comb_e * (gamma * y + ce_row)

    @pl.when(e == 0)
    def _init():
        out_ref[...] = contrib

    @pl.when(e != 0)
    def _acc():
        out_ref[...] += contrib


@jax.jit
def kernel(x, Ins_tk, Wg, We, be, Wgam, Wbeta, Wr):
    B, C, L = x.shape
    xf = x.reshape(-1, L)
    T = xf.shape[0]
    ins = Ins_tk[0]
    wg_pad = jnp.zeros((EPAD, L), jnp.float32).at[:E].set(Wg)
    wr_pad = jnp.zeros((EPAD, L), jnp.float32).at[:E].set(Wr)
    wgam_pad = jnp.zeros((EPAD, L), jnp.float32).at[:E].set(Wgam)

    out = pl.pallas_call(
        _moe_body,
        grid=(E,),
        in_specs=[
            pl.BlockSpec((T, L), lambda e: (0, 0)),
            pl.BlockSpec((EPAD, L), lambda e: (0, 0)),
            pl.BlockSpec((EPAD, L), lambda e: (0, 0)),
            pl.BlockSpec((EPAD, L), lambda e: (0, 0)),
            pl.BlockSpec(ins.shape, lambda e: (0, 0)),
            pl.BlockSpec((E, L), lambda e: (0, 0)),
            pl.BlockSpec((1, OUT_LEN, L), lambda e: (e, 0, 0)),
            pl.BlockSpec((1, OUT_LEN, L), lambda e: (e, 0, 0)),
        ],
        out_specs=pl.BlockSpec((T, OUT_LEN), lambda e: (0, 0)),
        out_shape=jax.ShapeDtypeStruct((T, OUT_LEN), jnp.float32),
        scratch_shapes=[
            pltpu.VMEM((T, 1), jnp.int32),
            pltpu.VMEM((T, 1), jnp.float32),
            pltpu.VMEM((1, EPAD), jnp.float32),
        ],
        compiler_params=pltpu.CompilerParams(
            dimension_semantics=("arbitrary",),
        ),
    )(xf, wg_pad, wr_pad, wgam_pad, ins, be, We, Wbeta)
    return out.reshape(B, C, OUT_LEN)
